# Initial kernel scaffold; baseline (speedup 1.0000x reference)
import jax
import jax.numpy as jnp
from jax.experimental import pallas as pl


def kernel(b, C, nb, nC, emb_weight, ctx_scheme, simi_kernel):
    raise NotImplementedError("write your pallas kernel here")


# speedup vs baseline: 1.5107x; 1.5107x over previous
import functools

import jax
import jax.numpy as jnp
from jax import lax
from jax.experimental import pallas as pl
from jax.experimental.pallas import tpu as pltpu
from jax.experimental.pallas import tpu_sc as plsc

EMBED = 64
CTX = 20
BATCH = 16384
L = 16
NC, NS = 2, 16
NW = NC * NS
BW = BATCH // NW
CHUNK = 16
NCHUNK = BW // CHUNK
CROWS = CHUNK * CTX
NQ = EMBED // L


def _chunk_copies(table, bidx, nbidx, cidx, ncidx, brows, nbrows, crows,
                  ncrows, g, sem):
  cps = [
      pltpu.make_async_copy(
          table.at[bidx.at[pl.ds(g * CHUNK, CHUNK)]], brows, sem),
      pltpu.make_async_copy(
          table.at[nbidx.at[pl.ds(g * CHUNK, CHUNK)]], nbrows, sem),
  ]
  for k in range(CROWS // 64):
    cps.append(pltpu.make_async_copy(
        table.at[cidx.at[pl.ds(g * CROWS + k * 64, 64)]],
        crows.at[pl.ds(k * 64, 64)], sem))
  for k in range(CROWS // 64):
    cps.append(pltpu.make_async_copy(
        table.at[ncidx.at[pl.ds(g * CROWS + k * 64, 64)]],
        ncrows.at[pl.ds(k * 64, 64)], sem))
  return cps


def _chunk_reduce(brows, nbrows, crows, ncrows, wv, acc0):
  def elem(i, acc):
    be = [brows[i, pl.ds(q * L, L)] for q in range(NQ)]
    nbe = [nbrows[i, pl.ds(q * L, L)] for q in range(NQ)]
    hc = [None] * NQ
    hn = [None] * NQ
    base = i * CTX
    for j in range(CTX):
      w = wv[j, :]
      for q in range(NQ):
        cr = w * crows[base + j, pl.ds(q * L, L)]
        nr = w * ncrows[base + j, pl.ds(q * L, L)]
        hc[q] = cr if j == 0 else hc[q] + cr
        hn[q] = nr if j == 0 else hn[q] + nr
    for q in range(NQ):
      acc = acc + (0.5 * nbe[q] - be[q]) * hc[q] + (0.5 * be[q]) * hn[q]
    return acc

  return lax.fori_loop(0, CHUNK, elem, acc0)


def _make_sc_kernel():
  mesh = plsc.VectorSubcoreMesh(core_axis_name="c", subcore_axis_name="s")

  @functools.partial(
      pl.kernel,
      mesh=mesh,
      out_type=jax.ShapeDtypeStruct(((NW + 1) * L,), jnp.float32),
      compiler_params=pltpu.CompilerParams(use_tc_tiling_on_sc=False),
      scratch_types=[
          pltpu.VMEM((BW,), jnp.int32),
          pltpu.VMEM((BW,), jnp.int32),
          pltpu.VMEM((BW * CTX,), jnp.int32),
          pltpu.VMEM((BW * CTX,), jnp.int32),
          pltpu.VMEM((CTX, L), jnp.float32),
          pltpu.VMEM((CHUNK, EMBED), jnp.float32),
          pltpu.VMEM((CHUNK, EMBED), jnp.float32),
          pltpu.VMEM((CHUNK, EMBED), jnp.float32),
          pltpu.VMEM((CHUNK, EMBED), jnp.float32),
          pltpu.VMEM((CROWS, EMBED), jnp.float32),
          pltpu.VMEM((CROWS, EMBED), jnp.float32),
          pltpu.VMEM((CROWS, EMBED), jnp.float32),
          pltpu.VMEM((CROWS, EMBED), jnp.float32),
          pltpu.VMEM((L,), jnp.float32),
          pltpu.VMEM((EMBED, EMBED), jnp.float32),
          pltpu.SemaphoreType.DMA,
          pltpu.SemaphoreType.DMA,
      ],
  )
  def sc_kernel(table, b, nb, cf, ncf, wvh, simi, out,
                bidx, nbidx, cidx, ncidx, wv,
                brows0, brows1, nbrows0, nbrows1,
                crows0, crows1, ncrows0, ncrows1,
                accv, simiv, sem0, sem1):
    wid = lax.axis_index("s") * NC + lax.axis_index("c")

    pltpu.sync_copy(b.at[pl.ds(wid * BW, BW)], bidx)
    pltpu.sync_copy(nb.at[pl.ds(wid * BW, BW)], nbidx)
    pltpu.sync_copy(cf.at[pl.ds(wid * BW * CTX, BW * CTX)], cidx)
    pltpu.sync_copy(ncf.at[pl.ds(wid * BW * CTX, BW * CTX)], ncidx)
    pltpu.sync_copy(wvh, wv)

    buf0 = (brows0, nbrows0, crows0, ncrows0)
    buf1 = (brows1, nbrows1, crows1, ncrows1)

    def copies(g, buf, sem):
      return _chunk_copies(table, bidx, nbidx, cidx, ncidx, *buf, g, sem)

    for cp in copies(0, buf0, sem0):
      cp.start()

    def outer(g2, acc):
      g = 2 * g2
      for cp in copies(g + 1, buf1, sem1):
        cp.start()
      for cp in copies(g, buf0, sem0):
        cp.wait()
      acc = _chunk_reduce(*buf0, wv, acc)

      @pl.when(g + 2 < NCHUNK)
      def _():
        for cp in copies(g + 2, buf0, sem0):
          cp.start()

      for cp in copies(g + 1, buf1, sem1):
        cp.wait()
      return _chunk_reduce(*buf1, wv, acc)

    acc = lax.fori_loop(0, NCHUNK // 2, outer,
                        jnp.zeros((L,), jnp.float32))
    accv[...] = acc
    pltpu.sync_copy(accv, out.at[pl.ds(wid * L, L)])

    @pl.when(wid == 0)
    def _():
      pltpu.sync_copy(simi, simiv)

      def row(r, a):
        for q in range(NQ):
          v = simiv[r, pl.ds(q * L, L)]
          a = a + v * v
        return a

      accv[...] = lax.fori_loop(0, EMBED, row, jnp.zeros((L,), jnp.float32))
      pltpu.sync_copy(accv, out.at[pl.ds(NW * L, L)])

  return sc_kernel


_sc_kernel = _make_sc_kernel()


def kernel(b, C, nb, nC, emb_weight, ctx_scheme, simi_kernel):
  h = jax.nn.softmax(ctx_scheme.astype(jnp.float32), axis=0)
  wv = jnp.broadcast_to(h[:, None], (CTX, L))
  parts = _sc_kernel(
      emb_weight,
      b.astype(jnp.int32),
      nb.astype(jnp.int32),
      C.reshape(-1).astype(jnp.int32),
      nC.reshape(-1).astype(jnp.int32),
      wv,
      simi_kernel,
  )
  return (jnp.sum(parts[:NW * L]) / BATCH
          + 0.1 * jnp.sum(parts[NW * L:]) / (EMBED * EMBED))
